# baseline (speedup 1.0000x reference)
import functools

import jax
import jax.numpy as jnp
from jax import lax
from jax.experimental import pallas as pl
from jax.experimental.pallas import tpu as pltpu
from jax.experimental.pallas import tpu_sc as plsc

V = 100000
D = 300
SEQ = 50
B = 4096

DP = 384

NC = 2
NS = 16
NW = NC * NS
BPW = B // NW
CHUNK = 64
NCHUNK = BPW // CHUNK
NJ = SEQ // 2
GR = 2 * CHUNK

LANES = 16

_sc_mesh = plsc.VectorSubcoreMesh(
    core_axis_name="c", subcore_axis_name="s", num_cores=NC, num_subcores=NS
)


@functools.partial(
    pl.kernel,
    out_type=(
        jax.ShapeDtypeStruct((B, DP), jnp.float32),
        jax.ShapeDtypeStruct((B, DP), jnp.float32),
    ),
    mesh=_sc_mesh,
    scratch_types=[
        pltpu.VMEM((NJ, GR), jnp.int32),
        [pltpu.VMEM((GR, DP), jnp.float32) for _ in range(2)],
        pltpu.VMEM((CHUNK, DP), jnp.float32),
        [pltpu.SemaphoreType.DMA for _ in range(2)],
    ],
)
def _sc_pool(inpr_hbm, lut_hbm, slut_hbm, e1_hbm, e2_hbm,
             idx_c, bufM, accM, sems):
    wid = lax.axis_index("s") * NC + lax.axis_index("c")
    zero = jnp.zeros((LANES,), jnp.float32)

    def issue(tbl, j, slot):
        pltpu.async_copy(tbl.at[idx_c.at[j]], bufM[slot], sems[slot])

    def wait(slot):
        pltpu.make_async_copy(lut_hbm.at[idx_c.at[0]], bufM[slot],
                              sems[slot]).wait()

    def accum(slot):
        def r_body(r, inner):
            for k in range(DP // LANES):
                o = pl.ds(k * LANES, LANES)
                x = bufM[slot][r, o] + bufM[slot][CHUNK + r, o]
                plsc.addupdate(accM.at[r, o], x)
            return inner
        lax.fori_loop(0, CHUNK, r_body, 0)

    def run_pass(tbl):
        issue(tbl, 0, 0)

        def zero_body(r, carry):
            for k in range(DP // LANES):
                accM[r, pl.ds(k * LANES, LANES)] = zero
            return carry

        lax.fori_loop(0, CHUNK, zero_body, 0)

        def j_body(j, carry):
            for par in range(2):
                jj = 2 * j + par

                @pl.when(jj + 1 < NJ)
                def _():
                    issue(tbl, jj + 1, (par + 1) % 2)

                wait(par)
                accum(par)
            return carry

        lax.fori_loop(0, NJ // 2, j_body, 0)
        wait(0)
        accum(0)

    for c in range(NCHUNK):
        base = wid * BPW + c * CHUNK
        pltpu.sync_copy(inpr_hbm.at[wid, c], idx_c)
        run_pass(lut_hbm)
        pltpu.sync_copy(accM, e1_hbm.at[pl.ds(base, CHUNK), :])
        run_pass(slut_hbm)
        pltpu.sync_copy(accM, e2_hbm.at[pl.ds(base, CHUNK), :])


MB = 512


def _mlp_body(e1_ref, e2_ref, w1a_ref, w1b_ref, b1_ref, w2_ref, b2_ref, out_ref):
    h = jnp.dot(e1_ref[...], w1a_ref[...], preferred_element_type=jnp.float32)
    h = h + jnp.dot(e2_ref[...], w1b_ref[...], preferred_element_type=jnp.float32)
    h = jnp.maximum(h + b1_ref[...], 0.0)
    out_ref[...] = jnp.dot(h, w2_ref[...], preferred_element_type=jnp.float32) + b2_ref[...]


_mlp = pl.pallas_call(
    _mlp_body,
    grid=(B // MB,),
    in_specs=[
        pl.BlockSpec((MB, DP), lambda i: (i, 0)),
        pl.BlockSpec((MB, DP), lambda i: (i, 0)),
        pl.BlockSpec((DP, 2 * D), lambda i: (0, 0)),
        pl.BlockSpec((DP, 2 * D), lambda i: (0, 0)),
        pl.BlockSpec((1, 2 * D), lambda i: (0, 0)),
        pl.BlockSpec((2 * D, 1), lambda i: (0, 0)),
        pl.BlockSpec((1, 1), lambda i: (0, 0)),
    ],
    out_specs=pl.BlockSpec((MB, 1), lambda i: (i, 0)),
    out_shape=jax.ShapeDtypeStruct((B, 1), jnp.float32),
)


def kernel(input, lut, static_lut, W1, b1, W2, b2):
    lut_p = jnp.pad(lut, ((0, 0), (0, DP - D)))
    slut_p = jnp.pad(static_lut, ((0, 0), (0, DP - D)))
    inpr = (input.reshape(NJ, 2, NW, NCHUNK, CHUNK)
            .transpose(2, 3, 0, 1, 4)
            .reshape(NW, NCHUNK, NJ, GR))
    e1, e2 = _sc_pool(inpr, lut_p, slut_p)
    w1a = jnp.pad(W1[:D], ((0, DP - D), (0, 0)))
    w1b = jnp.pad(W1[D:], ((0, DP - D), (0, 0)))
    out = _mlp(e1, e2, w1a, w1b, b1.reshape(1, 2 * D), W2, b2.reshape(1, 1))
    return out.reshape(B)


# speedup vs baseline: 1.8833x; 1.8833x over previous
import functools

import jax
import jax.numpy as jnp
from jax import lax
from jax.experimental import pallas as pl
from jax.experimental.pallas import tpu as pltpu
from jax.experimental.pallas import tpu_sc as plsc

V = 100000
D = 300
SEQ = 50
B = 4096

MAIN = 256
TAIL = D - MAIN
TP = 128

NC = 2
NS = 16
NW = NC * NS
BPW = B // NW
CHUNK = 64
NCHUNK = BPW // CHUNK
NJ = SEQ // 2
GR = 2 * CHUNK

LANES = 16

_sc_mesh = plsc.VectorSubcoreMesh(
    core_axis_name="c", subcore_axis_name="s", num_cores=NC, num_subcores=NS
)


@functools.partial(
    pl.kernel,
    out_type=(
        jax.ShapeDtypeStruct((B, MAIN), jnp.float32),
        jax.ShapeDtypeStruct((B, TP), jnp.float32),
    ),
    mesh=_sc_mesh,
    scratch_types=[
        pltpu.VMEM((NJ, GR), jnp.int32),
        [pltpu.VMEM((GR, MAIN), jnp.float32) for _ in range(2)],
        [pltpu.VMEM((GR, TP), jnp.float32) for _ in range(2)],
        pltpu.VMEM((CHUNK, MAIN), jnp.float32),
        pltpu.VMEM((CHUNK, TP), jnp.float32),
        [pltpu.SemaphoreType.DMA for _ in range(2)],
    ],
)
def _sc_pool(inpr_hbm, tbl_hbm, tail_hbm, em_hbm, et_hbm,
             idx_c, bufM, bufT, accM, accT, sems):
    wid = lax.axis_index("s") * NC + lax.axis_index("c")
    zero = jnp.zeros((LANES,), jnp.float32)

    def issue(j, slot):
        pltpu.async_copy(tbl_hbm.at[:, pl.ds(0, MAIN)].at[idx_c.at[j]],
                         bufM[slot], sems[slot])
        pltpu.async_copy(tail_hbm.at[idx_c.at[j]], bufT[slot], sems[slot])

    def wait(slot):
        pltpu.make_async_copy(tbl_hbm.at[:, pl.ds(0, MAIN)].at[idx_c.at[0]],
                              bufM[slot], sems[slot]).wait()
        pltpu.make_async_copy(tail_hbm.at[idx_c.at[0]],
                              bufT[slot], sems[slot]).wait()

    def accum(slot):
        def r_body(r, inner):
            for k in range(MAIN // LANES):
                o = pl.ds(k * LANES, LANES)
                x = bufM[slot][r, o] + bufM[slot][CHUNK + r, o]
                plsc.addupdate(accM.at[r, o], x)
            for k in range(TP // LANES):
                o = pl.ds(k * LANES, LANES)
                x = bufT[slot][r, o] + bufT[slot][CHUNK + r, o]
                plsc.addupdate(accT.at[r, o], x)
            return inner
        lax.fori_loop(0, CHUNK, r_body, 0)

    for c in range(NCHUNK):
        base = wid * BPW + c * CHUNK
        pltpu.sync_copy(inpr_hbm.at[wid, c], idx_c)
        issue(0, 0)

        def zero_body(r, carry):
            for k in range(MAIN // LANES):
                accM[r, pl.ds(k * LANES, LANES)] = zero
            for k in range(TP // LANES):
                accT[r, pl.ds(k * LANES, LANES)] = zero
            return carry

        lax.fori_loop(0, CHUNK, zero_body, 0)

        def j_body(j, carry):
            for par in range(2):
                jj = 2 * j + par

                @pl.when(jj + 1 < NJ)
                def _():
                    issue(jj + 1, (par + 1) % 2)

                wait(par)
                accum(par)
            return carry

        lax.fori_loop(0, NJ // 2, j_body, 0)
        wait(0)
        accum(0)

        pltpu.sync_copy(accM, em_hbm.at[pl.ds(base, CHUNK), :])
        pltpu.sync_copy(accT, et_hbm.at[pl.ds(base, CHUNK), :])


MB = 512


def _mlp_body(e1m_ref, e1t_ref, e2m_ref, e2t_ref,
              w1a_ref, w1at_ref, w1b_ref, w1bt_ref,
              b1_ref, w2_ref, b2_ref, out_ref):
    h = jnp.dot(e1m_ref[...], w1a_ref[...], preferred_element_type=jnp.float32)
    h = h + jnp.dot(e1t_ref[...], w1at_ref[...], preferred_element_type=jnp.float32)
    h = h + jnp.dot(e2m_ref[...], w1b_ref[...], preferred_element_type=jnp.float32)
    h = h + jnp.dot(e2t_ref[...], w1bt_ref[...], preferred_element_type=jnp.float32)
    h = jnp.maximum(h + b1_ref[...], 0.0)
    out_ref[...] = jnp.dot(h, w2_ref[...], preferred_element_type=jnp.float32) + b2_ref[...]


_mlp = pl.pallas_call(
    _mlp_body,
    grid=(B // MB,),
    in_specs=[
        pl.BlockSpec((MB, MAIN), lambda i: (i, 0)),
        pl.BlockSpec((MB, TP), lambda i: (i, 0)),
        pl.BlockSpec((MB, MAIN), lambda i: (i, 0)),
        pl.BlockSpec((MB, TP), lambda i: (i, 0)),
        pl.BlockSpec((MAIN, 2 * D), lambda i: (0, 0)),
        pl.BlockSpec((TP, 2 * D), lambda i: (0, 0)),
        pl.BlockSpec((MAIN, 2 * D), lambda i: (0, 0)),
        pl.BlockSpec((TP, 2 * D), lambda i: (0, 0)),
        pl.BlockSpec((1, 2 * D), lambda i: (0, 0)),
        pl.BlockSpec((2 * D, 1), lambda i: (0, 0)),
        pl.BlockSpec((1, 1), lambda i: (0, 0)),
    ],
    out_specs=pl.BlockSpec((MB, 1), lambda i: (i, 0)),
    out_shape=jax.ShapeDtypeStruct((B, 1), jnp.float32),
)


def _padtail(t):
    return jnp.pad(t[:, MAIN:], ((0, 0), (0, TP - TAIL)))


def kernel(input, lut, static_lut, W1, b1, W2, b2):
    inpr = (input.reshape(NJ, 2, NW, NCHUNK, CHUNK)
            .transpose(2, 3, 0, 1, 4)
            .reshape(NW, NCHUNK, NJ, GR))
    e1m, e1t = _sc_pool(inpr, lut, _padtail(lut))
    e2m, e2t = _sc_pool(inpr, static_lut, _padtail(static_lut))
    w1a = W1[:MAIN]
    w1at = jnp.pad(W1[MAIN:D], ((0, TP - TAIL), (0, 0)))
    w1b = W1[D:D + MAIN]
    w1bt = jnp.pad(W1[D + MAIN:], ((0, TP - TAIL), (0, 0)))
    out = _mlp(e1m, e1t, e2m, e2t, w1a, w1at, w1b, w1bt,
               b1.reshape(1, 2 * D), W2, b2.reshape(1, 1))
    return out.reshape(B)
